# baseline (speedup 1.0000x reference)
import jax
import jax.numpy as jnp
from jax import lax
from jax.experimental import pallas as pl
from jax.experimental.pallas import tpu as pltpu
from jax.experimental.pallas import tpu_sc as plsc

_BATCH = 16384
_IN = 128
_OUT = 64
_ROWS = 1000000
_NC = 2
_NS = 16
_NW = _NC * _NS
_BPW = _BATCH // _NW

_TBLK = 40960
_NBLK = (_ROWS + _TBLK - 1) // _TBLK
_WROWS = _NBLK * (_TBLK // 2)


def _tc_relayout(embT):
    eye = jnp.eye(_OUT, dtype=jnp.float32)

    def body(in_ref, eye_ref, out_ref):
        t = in_ref[...].T
        out_ref[:, :_OUT] = t[: _TBLK // 2, :]
        out_ref[:, _OUT:] = t[_TBLK // 2:, :]

    return pl.pallas_call(
        body,
        grid=(_NBLK,),
        in_specs=[
            pl.BlockSpec((_OUT, _TBLK), lambda i: (0, i)),
            pl.BlockSpec((_OUT, _OUT), lambda i: (0, 0)),
        ],
        out_specs=pl.BlockSpec((_TBLK // 2, 2 * _OUT), lambda i: (i, 0)),
        out_shape=jax.ShapeDtypeStruct((_WROWS, 2 * _OUT), jnp.float32),
        compiler_params=pltpu.CompilerParams(
            dimension_semantics=("parallel",)
        ),
    )(embT, eye)


def _sc_gather(table, idx):
    mesh = plsc.VectorSubcoreMesh(core_axis_name="c", subcore_axis_name="s")

    @pl.kernel(
        out_type=jax.ShapeDtypeStruct((_BATCH, 2 * _OUT), jnp.float32),
        mesh=mesh,
        scratch_types=[
            pltpu.VMEM((_BPW,), jnp.int32),
            pltpu.VMEM((_BPW, 2 * _OUT), jnp.float32),
            pltpu.SemaphoreType.DMA,
        ],
        compiler_params=pltpu.CompilerParams(use_tc_tiling_on_sc=False),
    )
    def gather_kernel(emb_hbm, idx_hbm, out_hbm, idx_v, rows_v, sem):
        wid = lax.axis_index("s") * _NC + lax.axis_index("c")
        base = wid * _BPW
        pltpu.sync_copy(idx_hbm.at[pl.ds(base, _BPW)], idx_v)
        pltpu.async_copy(emb_hbm.at[idx_v], rows_v, sem).wait()
        pltpu.sync_copy(rows_v, out_hbm.at[pl.ds(base, _BPW)])

    return gather_kernel(table, idx)


def _tc_fused(x, W_fc, g, half):
    blk = 2048

    def body(x_ref, wfc_ref, g_ref, p_ref, o_ref):
        h = jax.lax.dot_general(
            x_ref[...],
            wfc_ref[...],
            (((1,), (1,)), ((), ())),
            preferred_element_type=jnp.float32,
        )
        g = g_ref[...]
        w_i = jnp.where(p_ref[...] == 0, g[:, :_OUT], g[:, _OUT:])
        o_ref[...] = h * w_i

    return pl.pallas_call(
        body,
        grid=(_BATCH // blk,),
        in_specs=[
            pl.BlockSpec((blk, _IN), lambda i: (i, 0)),
            pl.BlockSpec((_OUT, _IN), lambda i: (0, 0)),
            pl.BlockSpec((blk, 2 * _OUT), lambda i: (i, 0)),
            pl.BlockSpec((blk, 1), lambda i: (i, 0)),
        ],
        out_specs=pl.BlockSpec((blk, _OUT), lambda i: (i, 0)),
        out_shape=jax.ShapeDtypeStruct((_BATCH, _OUT), jnp.float32),
    )(x, W_fc, g, half)


def kernel(x, id, W_fc, emb):
    idx = id.astype(jnp.int32)
    table = _tc_relayout(emb.T)
    wide_idx = (idx // _TBLK) * (_TBLK // 2) + (idx % (_TBLK // 2))
    half = ((idx // (_TBLK // 2)) & 1).reshape(_BATCH, 1)
    g = _sc_gather(table, wide_idx)
    return _tc_fused(x, W_fc, g, half)


# speedup vs baseline: 1.0169x; 1.0169x over previous
import jax
import jax.numpy as jnp
from jax import lax
from jax.experimental import pallas as pl
from jax.experimental.pallas import tpu as pltpu
from jax.experimental.pallas import tpu_sc as plsc

_BATCH = 16384
_IN = 128
_OUT = 64
_ROWS = 1000000
_NC = 2
_NS = 16
_NW = _NC * _NS
_BPW = _BATCH // _NW

_TBLK = 32768
_NBLK = (_ROWS + _TBLK - 1) // _TBLK
_WROWS = _NBLK * (_TBLK // 2)


def _tc_relayout(embT):
    def body(in_ref, out_ref):
        t = in_ref[...].T
        out_ref[:, :_OUT] = t[: _TBLK // 2, :]
        out_ref[:, _OUT:] = t[_TBLK // 2:, :]

    return pl.pallas_call(
        body,
        grid=(_NBLK,),
        in_specs=[
            pl.BlockSpec((_OUT, _TBLK), lambda i: (0, i)),
        ],
        out_specs=pl.BlockSpec((_TBLK // 2, 2 * _OUT), lambda i: (i, 0)),
        out_shape=jax.ShapeDtypeStruct((_WROWS, 2 * _OUT), jnp.float32),
    )(embT)


def _sc_gather(table, idx):
    mesh = plsc.VectorSubcoreMesh(core_axis_name="c", subcore_axis_name="s")

    @pl.kernel(
        out_type=jax.ShapeDtypeStruct((_BATCH, 2 * _OUT), jnp.float32),
        mesh=mesh,
        scratch_types=[
            pltpu.VMEM((_BPW,), jnp.int32),
            pltpu.VMEM((_BPW, 2 * _OUT), jnp.float32),
            pltpu.SemaphoreType.DMA,
        ],
        compiler_params=pltpu.CompilerParams(use_tc_tiling_on_sc=False),
    )
    def gather_kernel(emb_hbm, idx_hbm, out_hbm, idx_v, rows_v, sem):
        wid = lax.axis_index("s") * _NC + lax.axis_index("c")
        base = wid * _BPW
        pltpu.sync_copy(idx_hbm.at[pl.ds(base, _BPW)], idx_v)
        pltpu.async_copy(emb_hbm.at[idx_v], rows_v, sem).wait()
        pltpu.sync_copy(rows_v, out_hbm.at[pl.ds(base, _BPW)])

    return gather_kernel(table, idx)


def _tc_fused(x, W_fc, g, half):
    blk = 2048

    def body(x_ref, wfc_ref, g_ref, p_ref, o_ref):
        h = jax.lax.dot_general(
            x_ref[...],
            wfc_ref[...],
            (((1,), (1,)), ((), ())),
            preferred_element_type=jnp.float32,
        )
        g = g_ref[...]
        w_i = jnp.where(p_ref[...] == 0, g[:, :_OUT], g[:, _OUT:])
        o_ref[...] = h * w_i

    return pl.pallas_call(
        body,
        grid=(_BATCH // blk,),
        in_specs=[
            pl.BlockSpec((blk, _IN), lambda i: (i, 0)),
            pl.BlockSpec((_OUT, _IN), lambda i: (0, 0)),
            pl.BlockSpec((blk, 2 * _OUT), lambda i: (i, 0)),
            pl.BlockSpec((blk, 1), lambda i: (i, 0)),
        ],
        out_specs=pl.BlockSpec((blk, _OUT), lambda i: (i, 0)),
        out_shape=jax.ShapeDtypeStruct((_BATCH, _OUT), jnp.float32),
    )(x, W_fc, g, half)


def kernel(x, id, W_fc, emb):
    idx = id.astype(jnp.int32)
    table = _tc_relayout(emb.T)
    wide_idx = (idx // _TBLK) * (_TBLK // 2) + (idx % (_TBLK // 2))
    half = ((idx // (_TBLK // 2)) & 1).reshape(_BATCH, 1)
    g = _sc_gather(table, wide_idx)
    return _tc_fused(x, W_fc, g, half)


# speedup vs baseline: 1.0228x; 1.0057x over previous
import jax
import jax.numpy as jnp
from jax import lax
from jax.experimental import pallas as pl
from jax.experimental.pallas import tpu as pltpu
from jax.experimental.pallas import tpu_sc as plsc

_BATCH = 16384
_IN = 128
_OUT = 64
_ROWS = 1000000
_NC = 2
_NS = 16
_NW = _NC * _NS
_BPW = _BATCH // _NW

_TBLK = 32768
_NBLK = (_ROWS + _TBLK - 1) // _TBLK
_WROWS = _NBLK * (_TBLK // 2)


def _tc_relayout(embT):
    def body(in_ref, out_ref):
        t = in_ref[...].T
        out_ref[:, :_OUT] = t[: _TBLK // 2, :]
        out_ref[:, _OUT:] = t[_TBLK // 2:, :]

    return pl.pallas_call(
        body,
        grid=(_NBLK,),
        in_specs=[
            pl.BlockSpec((_OUT, _TBLK), lambda i: (0, i)),
        ],
        out_specs=pl.BlockSpec((_TBLK // 2, 2 * _OUT), lambda i: (i, 0)),
        out_shape=jax.ShapeDtypeStruct((_WROWS, 2 * _OUT), jnp.float32),
    )(embT)


def _sc_gather(table, idx):
    mesh = plsc.VectorSubcoreMesh(core_axis_name="c", subcore_axis_name="s")

    @pl.kernel(
        out_type=jax.ShapeDtypeStruct((_BATCH, 2 * _OUT), jnp.float32),
        mesh=mesh,
        scratch_types=[
            pltpu.VMEM((_BPW,), jnp.int32),
            pltpu.VMEM((_BPW, 2 * _OUT), jnp.float32),
            pltpu.SemaphoreType.DMA,
        ],
        compiler_params=pltpu.CompilerParams(use_tc_tiling_on_sc=False),
    )
    def gather_kernel(emb_hbm, idx_hbm, out_hbm, idx_v, rows_v, sem):
        wid = lax.axis_index("s") * _NC + lax.axis_index("c")
        base = wid * _BPW
        pltpu.sync_copy(idx_hbm.at[pl.ds(base, _BPW)], idx_v)
        pltpu.async_copy(emb_hbm.at[idx_v], rows_v, sem).wait()
        pltpu.sync_copy(rows_v, out_hbm.at[pl.ds(base, _BPW)])

    return gather_kernel(table, idx)


def _tc_fused(x, W_fc, g, half):
    blk = 4096

    def body(x_ref, wfc_ref, g_ref, p_ref, o_ref):
        h = jax.lax.dot_general(
            x_ref[...],
            wfc_ref[...],
            (((1,), (1,)), ((), ())),
            preferred_element_type=jnp.float32,
        )
        g = g_ref[...]
        w_i = jnp.where(p_ref[...] == 0, g[:, :_OUT], g[:, _OUT:])
        o_ref[...] = h * w_i

    return pl.pallas_call(
        body,
        grid=(_BATCH // blk,),
        in_specs=[
            pl.BlockSpec((blk, _IN), lambda i: (i, 0)),
            pl.BlockSpec((_OUT, _IN), lambda i: (0, 0)),
            pl.BlockSpec((blk, 2 * _OUT), lambda i: (i, 0)),
            pl.BlockSpec((blk, 1), lambda i: (i, 0)),
        ],
        out_specs=pl.BlockSpec((blk, _OUT), lambda i: (i, 0)),
        out_shape=jax.ShapeDtypeStruct((_BATCH, _OUT), jnp.float32),
    )(x, W_fc, g, half)


def kernel(x, id, W_fc, emb):
    idx = id.astype(jnp.int32)
    table = _tc_relayout(emb.T)
    wide_idx = (idx // _TBLK) * (_TBLK // 2) + (idx % (_TBLK // 2))
    half = ((idx // (_TBLK // 2)) & 1).reshape(_BATCH, 1)
    g = _sc_gather(table, wide_idx)
    return _tc_fused(x, W_fc, g, half)
